# baseline (speedup 1.0000x reference)
import functools

import jax
import jax.numpy as jnp
from jax.experimental import pallas as pl
from jax.experimental.pallas import tpu as pltpu

_NUM_BINS = 8


def _fused_kernel(x_ref, top_ref, bot_ref, co_ref, w_ref, out_ref,
                  cache_ref, scr_ref, mm_ref, *, hb, nchunks, ncache, himg):
    p = pl.program_id(0)
    b = pl.program_id(1)
    i = pl.program_id(2)
    wdim = x_ref.shape[2]
    cdim = x_ref.shape[3]

    @pl.when(p == 0)
    def _reduce_sweep():
        xb = x_ref[0]
        m1 = jnp.min(xb)
        m2 = jnp.max(xb)
        first = jnp.logical_and(b == 0, i == 0)

        @pl.when(first)
        def _init():
            mm_ref[0] = m1
            mm_ref[1] = m2

        @pl.when(jnp.logical_not(first))
        def _acc():
            mm_ref[0] = jnp.minimum(mm_ref[0], m1)
            mm_ref[1] = jnp.maximum(mm_ref[1], m2)

        @pl.when(b < ncache)
        def _fill_cache():
            cache_ref[b, pl.ds(i * hb, hb)] = xb

    def conv_body(get_row):
        zrow = jnp.zeros((1, cdim), jnp.float32)
        for r in range(hb + 2):
            scr_ref[r] = jnp.concatenate([zrow, get_row(r), zrow], axis=0)
        wk = w_ref[...]
        for h in range(hb):
            acc = None
            for dh in range(3):
                for dw in range(3):
                    t = scr_ref[h + dh, dw:dw + wdim, :] * wk[dh, dw, :]
                    acc = t if acc is None else acc + t
            out_ref[0, h] = acc

    def make_scale():
        mn = mm_ref[0]
        mx = mm_ref[1]
        binw = (mx - mn + 1e-8) / float(_NUM_BINS)
        co = co_ref[...]

        def scale(v):
            sel = jnp.broadcast_to(co[0, :], v.shape)
            for k in range(1, _NUM_BINS):
                sel = jnp.where(v >= mn + float(k) * binw, co[k, :], sel)
            return v * sel

        return scale

    @pl.when(jnp.logical_and(p == 1, b < ncache))
    def _conv_cached():
        scale = make_scale()
        base = i * hb

        def get_row(r):
            if r == 0:
                row = cache_ref[b, jnp.maximum(base - 1, 0)]
                return scale(row) * jnp.where(i > 0, 1.0, 0.0)
            if r == hb + 1:
                row = cache_ref[b, jnp.minimum(base + hb, himg - 1)]
                return scale(row) * jnp.where(i < nchunks - 1, 1.0, 0.0)
            return scale(cache_ref[b, base + (r - 1)])

        conv_body(get_row)

    @pl.when(jnp.logical_and(p == 1, b >= ncache))
    def _conv_streamed():
        scale = make_scale()

        def get_row(r):
            if r == 0:
                return scale(top_ref[0, 0]) * jnp.where(i > 0, 1.0, 0.0)
            if r == hb + 1:
                return scale(bot_ref[0, 0]) * jnp.where(i < nchunks - 1,
                                                        1.0, 0.0)
            return scale(x_ref[0, r - 1])

        conv_body(get_row)


def kernel(x, co_matrix, w_spatial):
    b, h, w, c = x.shape
    hb = 16 if h % 16 == 0 else h // 4
    nchunks = h // hb
    ncache = b // 2
    last_b = b - 1
    last_i = nchunks - 1

    def x_map(p, bi, i):
        parked = jnp.logical_and(p == 1, bi < ncache)
        return (jnp.where(parked, last_b, bi),
                jnp.where(parked, last_i, i), 0, 0)

    def top_map(p, bi, i):
        active = jnp.logical_and(p == 1, bi >= ncache)
        return (jnp.where(active, bi, ncache),
                jnp.where(active, jnp.maximum(i * hb - 1, 0), 0), 0, 0)

    def bot_map(p, bi, i):
        active = jnp.logical_and(p == 1, bi >= ncache)
        return (jnp.where(active, bi, ncache),
                jnp.where(active, jnp.minimum((i + 1) * hb, h - 1), 0), 0, 0)

    def out_map(p, bi, i):
        return (jnp.where(p == 1, bi, 0), jnp.where(p == 1, i, 0), 0, 0)

    return pl.pallas_call(
        functools.partial(_fused_kernel, hb=hb, nchunks=nchunks,
                          ncache=ncache, himg=h),
        grid=(2, b, nchunks),
        in_specs=[
            pl.BlockSpec((1, hb, w, c), x_map),
            pl.BlockSpec((1, 1, w, c), top_map),
            pl.BlockSpec((1, 1, w, c), bot_map),
            pl.BlockSpec((_NUM_BINS, c), lambda p, bi, i: (0, 0)),
            pl.BlockSpec((3, 3, c), lambda p, bi, i: (0, 0, 0)),
        ],
        out_specs=pl.BlockSpec((1, hb, w, c), out_map),
        out_shape=jax.ShapeDtypeStruct((b, h, w, c), x.dtype),
        scratch_shapes=[
            pltpu.VMEM((ncache, h, w, c), jnp.float32),
            pltpu.VMEM((hb + 2, w + 2, c), jnp.float32),
            pltpu.SMEM((2,), jnp.float32),
        ],
        compiler_params=pltpu.CompilerParams(
            dimension_semantics=("arbitrary", "arbitrary", "arbitrary"),
            vmem_limit_bytes=67108864),
    )(x, x, x, co_matrix, w_spatial)


# speedup vs baseline: 1.0040x; 1.0040x over previous
import functools

import jax
import jax.numpy as jnp
from jax.experimental import pallas as pl
from jax.experimental.pallas import tpu as pltpu

_NUM_BINS = 8
_CACHE_BYTES = 48 * 1024 * 1024


def _fused_kernel(x_ref, top_ref, bot_ref, co_ref, w_ref, out_ref,
                  cache_ref, scr_ref, mm_ref, *, hb, nchunks, cached_chunks):
    p = pl.program_id(0)
    b = pl.program_id(1)
    i = pl.program_id(2)
    k = b * nchunks + i
    wdim = x_ref.shape[2]
    cdim = x_ref.shape[3]

    @pl.when(p == 0)
    def _reduce_sweep():
        xb = x_ref[0]
        m1 = jnp.min(xb)
        m2 = jnp.max(xb)
        first = jnp.logical_and(b == 0, i == 0)

        @pl.when(first)
        def _init():
            mm_ref[0] = m1
            mm_ref[1] = m2

        @pl.when(jnp.logical_not(first))
        def _acc():
            mm_ref[0] = jnp.minimum(mm_ref[0], m1)
            mm_ref[1] = jnp.maximum(mm_ref[1], m2)

        @pl.when(k < cached_chunks)
        def _fill_cache():
            cache_ref[pl.ds(k * hb, hb)] = xb

    def make_scale():
        mn = mm_ref[0]
        mx = mm_ref[1]
        binw = (mx - mn + 1e-8) / float(_NUM_BINS)
        co = co_ref[...]

        def scale(v):
            sel = jnp.broadcast_to(co[0, :], v.shape)
            for j in range(1, _NUM_BINS):
                sel = jnp.where(v >= mn + float(j) * binw, co[j, :], sel)
            return v * sel

        return scale

    def conv_body(mid_row):
        scale = make_scale()

        def get_row(r):
            if r == 0:
                return scale(top_ref[0, 0]) * jnp.where(i > 0, 1.0, 0.0)
            if r == hb + 1:
                return scale(bot_ref[0, 0]) * jnp.where(i < nchunks - 1,
                                                        1.0, 0.0)
            return scale(mid_row(r - 1))

        zrow = jnp.zeros((1, cdim), jnp.float32)

        def stage(r):
            scr_ref[r % 4] = jnp.concatenate([zrow, get_row(r), zrow], axis=0)

        stage(0)
        stage(1)
        wk = w_ref[...]
        for h in range(hb):
            stage(h + 2)
            acc = None
            for dh in range(3):
                for dw in range(3):
                    t = scr_ref[(h + dh) % 4, dw:dw + wdim, :] * wk[dh, dw, :]
                    acc = t if acc is None else acc + t
            out_ref[0, h] = acc

    @pl.when(jnp.logical_and(p == 1, k < cached_chunks))
    def _conv_cached():
        conv_body(lambda r: cache_ref[k * hb + r])

    @pl.when(jnp.logical_and(p == 1, k >= cached_chunks))
    def _conv_streamed():
        conv_body(lambda r: x_ref[0, r])


def kernel(x, co_matrix, w_spatial):
    b, h, w, c = x.shape
    hb = 28 if h % 28 == 0 else h // 4
    nchunks = h // hb
    total_chunks = b * nchunks
    cpad = -(-c // 128) * 128
    cache_rows_fit = _CACHE_BYTES // (w * cpad * 4) // hb * hb
    cached_chunks = min(total_chunks, cache_rows_fit // hb)
    cache_rows = max(cached_chunks * hb, hb)
    last_b = b - 1
    last_i = nchunks - 1

    def x_map(p, bi, i):
        parked = jnp.logical_and(p == 1,
                                 bi * nchunks + i < cached_chunks)
        return (jnp.where(parked, last_b, bi),
                jnp.where(parked, last_i, i), 0, 0)

    def top_map(p, bi, i):
        return (jnp.where(p == 1, bi, 0),
                jnp.where(p == 1, jnp.maximum(i * hb - 1, 0), 0), 0, 0)

    def bot_map(p, bi, i):
        return (jnp.where(p == 1, bi, 0),
                jnp.where(p == 1, jnp.minimum((i + 1) * hb, h - 1), 0), 0, 0)

    def out_map(p, bi, i):
        return (jnp.where(p == 1, bi, 0), jnp.where(p == 1, i, 0), 0, 0)

    return pl.pallas_call(
        functools.partial(_fused_kernel, hb=hb, nchunks=nchunks,
                          cached_chunks=cached_chunks),
        grid=(2, b, nchunks),
        in_specs=[
            pl.BlockSpec((1, hb, w, c), x_map),
            pl.BlockSpec((1, 1, w, c), top_map),
            pl.BlockSpec((1, 1, w, c), bot_map),
            pl.BlockSpec((_NUM_BINS, c), lambda p, bi, i: (0, 0)),
            pl.BlockSpec((3, 3, c), lambda p, bi, i: (0, 0, 0)),
        ],
        out_specs=pl.BlockSpec((1, hb, w, c), out_map),
        out_shape=jax.ShapeDtypeStruct((b, h, w, c), x.dtype),
        scratch_shapes=[
            pltpu.VMEM((cache_rows, w, c), jnp.float32),
            pltpu.VMEM((4, w + 2, c), jnp.float32),
            pltpu.SMEM((2,), jnp.float32),
        ],
        compiler_params=pltpu.CompilerParams(
            dimension_semantics=("arbitrary", "arbitrary", "arbitrary"),
            vmem_limit_bytes=67108864),
    )(x, x, x, co_matrix, w_spatial)
